# baseline (speedup 1.0000x reference)
import functools

import jax
import numpy as np
import jax.numpy as jnp
from jax import lax
from jax.experimental import pallas as pl
from jax.experimental.pallas import tpu as pltpu
from jax.experimental.pallas import tpu_sc as plsc

L = 2048
H = 16
D = 64
U = 40
UP = 48
DH = 32
CHUNK = 128
QPT = 1024
NEG = np.float32(-3.0e38)


def _sc_stage1(qs, ks, idxp):
    mesh = plsc.VectorSubcoreMesh(core_axis_name="c", subcore_axis_name="s")

    @functools.partial(
        pl.kernel,
        mesh=mesh,
        compiler_params=pltpu.CompilerParams(needs_layout_passes=False),
        out_type=jax.ShapeDtypeStruct((H, L), jnp.float32),
        scratch_types=[
            pltpu.VMEM((L * DH,), jnp.float32),
            pltpu.VMEM((CHUNK * DH,), jnp.float32),
            pltpu.VMEM((CHUNK * UP,), jnp.int32),
            pltpu.VMEM((QPT * UP,), jnp.float32),
            pltpu.VMEM((QPT,), jnp.float32),
        ],
    )
    def sc_kernel(qT_hbm, kT_hbm, idx_hbm, m_hbm, kblk, qblk, idxblk, acc, mtile):
        h = lax.axis_index("s")
        half = lax.axis_index("c")
        q0 = half * QPT
        iota16 = lax.iota(jnp.int32, 16)

        def dh_body(dh, _):
            pltpu.sync_copy(kT_hbm.at[dh, h], kblk)

            def chunk_body(c, _):
                r0 = q0 + c * CHUNK
                pltpu.sync_copy(qT_hbm.at[dh, h, pl.ds(r0 * DH, CHUNK * DH)], qblk)
                pltpu.sync_copy(idx_hbm.at[pl.ds(r0 * UP, CHUNK * UP)], idxblk)

                def q_body(i, _):
                    a = c * CHUNK + i
                    for sv in range(3):
                        scol = iota16 + (sv * 16)
                        idxv = plsc.load_gather(idxblk, [scol + i * UP])
                        prev = plsc.load_gather(acc, [scol + a * UP])
                        av = jnp.where(dh == 0, jnp.zeros((16,), jnp.float32), prev)
                        kbase = idxv * DH
                        for d in range(DH):
                            kv = plsc.load_gather(kblk, [kbase + d])
                            qv = plsc.load_gather(qblk, [jnp.full((16,), i * DH + d, jnp.int32)])
                            av = av + kv * qv
                        plsc.store_scatter(acc, [scol + a * UP], av)
                    return 0

                lax.fori_loop(0, CHUNK, q_body, 0)
                return 0

            lax.fori_loop(0, QPT // CHUNK, chunk_body, 0)
            return 0

        lax.fori_loop(0, D // DH, dh_body, 0)

        def fin_body(g, _):
            rows = iota16 + g * 16
            base = rows * UP
            mx = jnp.full((16,), NEG, jnp.float32)
            sm = jnp.zeros((16,), jnp.float32)
            for s in range(U):
                vals = plsc.load_gather(acc, [base + s])
                mx = jnp.maximum(mx, vals)
                sm = sm + vals
            plsc.store_scatter(mtile, [rows], mx - sm * np.float32(1.0 / L))
            return 0

        lax.fori_loop(0, QPT // 16, fin_body, 0)
        pltpu.sync_copy(mtile, m_hbm.at[h, pl.ds(q0, QPT)])

    return sc_kernel(qs, ks, idxp)


def _tc_body(m_ref, q_ref, k_ref, v_ref, out_ref, qred, upds, ctx, idxs):
    mrow = m_ref[0]
    iota = lax.broadcasted_iota(jnp.int32, (1, L), 1)
    qred[pl.ds(U, 8), :] = jnp.zeros((8, D), jnp.float32)

    def tk(t, mcur):
        mx = jnp.max(mcur)
        it = jnp.min(jnp.where(mcur == mx, iota, L)).astype(jnp.int32)
        idxs[t] = it
        qred[pl.ds(t, 1), :] = q_ref[0, pl.ds(it, 1), :]
        return jnp.where(iota == it, NEG, mcur)

    lax.fori_loop(0, U, tk, mrow)

    kk = k_ref[0]
    scores = lax.dot_general(
        qred[...].astype(jnp.bfloat16), kk.astype(jnp.bfloat16),
        (((1,), (1,)), ((), ())),
        preferred_element_type=jnp.float32,
    ) * np.float32(0.125)
    smax = jnp.max(scores, axis=1, keepdims=True)
    e = jnp.exp(scores - smax)
    attn = e / jnp.sum(e, axis=1, keepdims=True)
    vv = v_ref[0]
    upds[...] = lax.dot_general(
        attn.astype(jnp.bfloat16), vv.astype(jnp.bfloat16),
        (((1,), (0,)), ((), ())),
        preferred_element_type=jnp.float32,
    )
    vmean = jnp.mean(vv, axis=0, keepdims=True)
    ctx[...] = jnp.broadcast_to(vmean, (U + 8, D))

    def sc(k, _):
        slot = jnp.minimum(idxs[k], U - 1)
        ctx[pl.ds(slot, 1), :] = upds[pl.ds(k, 1), :]
        return 0

    lax.fori_loop(0, U, sc, 0)
    out_ref[0] = ctx[0:U, :]


def _tc_stage2(m, qT, kT, vT):
    return pl.pallas_call(
        _tc_body,
        grid=(H,),
        in_specs=[
            pl.BlockSpec((1, 1, L), lambda h: (h, 0, 0)),
            pl.BlockSpec((1, L, D), lambda h: (h, 0, 0)),
            pl.BlockSpec((1, L, D), lambda h: (h, 0, 0)),
            pl.BlockSpec((1, L, D), lambda h: (h, 0, 0)),
        ],
        out_specs=pl.BlockSpec((1, U, D), lambda h: (h, 0, 0)),
        out_shape=jax.ShapeDtypeStruct((H, U, D), jnp.float32),
        scratch_shapes=[
            pltpu.VMEM((U + 8, D), jnp.float32),
            pltpu.VMEM((U + 8, D), jnp.float32),
            pltpu.VMEM((U + 8, D), jnp.float32),
            pltpu.SMEM((U,), jnp.int32),
        ],
    )(m, qT, kT, vT)


def kernel(queries, keys, values, attn_mask):
    qT = jnp.transpose(queries[0], (1, 0, 2)).astype(jnp.bfloat16).astype(jnp.float32)
    kT = jnp.transpose(keys[0], (1, 0, 2)).astype(jnp.bfloat16).astype(jnp.float32)
    vT = jnp.transpose(values[0], (1, 0, 2))
    idx = jax.random.randint(jax.random.key(42), (L, U), 0, L)
    idxp = jnp.concatenate(
        [idx, jnp.zeros((L, UP - U), idx.dtype)], axis=1
    ).astype(jnp.int32)
    qs = jnp.transpose(qT.reshape(H, L, D // DH, DH), (2, 0, 1, 3)).reshape(D // DH, H, L * DH)
    ks = jnp.transpose(kT.reshape(H, L, D // DH, DH), (2, 0, 1, 3)).reshape(D // DH, H, L * DH)
    m = _sc_stage1(qs, ks, idxp.reshape(L * UP))
    out3 = _tc_stage2(m.reshape(H, 1, L), qT, kT, vT)
    return (jnp.transpose(out3, (1, 0, 2))[None], None)


# speedup vs baseline: 1.0669x; 1.0669x over previous
import functools

import jax
import numpy as np
import jax.numpy as jnp
from jax import lax
from jax.experimental import pallas as pl
from jax.experimental.pallas import tpu as pltpu
from jax.experimental.pallas import tpu_sc as plsc

L = 2048
H = 16
D = 64
U = 40
UP = 48
DH = 32
CHUNK = 256
QPT = 1024
NEG = np.float32(-3.0e38)


DP = D // 2


def _sc_stage1(qpk, kpk, idxp):
    mesh = plsc.VectorSubcoreMesh(core_axis_name="c", subcore_axis_name="s")

    @functools.partial(
        pl.kernel,
        mesh=mesh,
        compiler_params=pltpu.CompilerParams(needs_layout_passes=False),
        out_type=jax.ShapeDtypeStruct((H, L), jnp.float32),
        scratch_types=[
            pltpu.VMEM((L * DP,), jnp.int32),
            pltpu.VMEM((CHUNK * DP,), jnp.int32),
            pltpu.VMEM((CHUNK * UP,), jnp.int32),
            pltpu.VMEM((CHUNK * UP,), jnp.float32),
            pltpu.VMEM((QPT,), jnp.float32),
        ],
    )
    def sc_kernel(qpk_hbm, kpk_hbm, idx_hbm, m_hbm, kblk, qblk, idxblk, acc, mtile):
        h = lax.axis_index("s")
        half = lax.axis_index("c")
        q0 = half * QPT
        iota16 = lax.iota(jnp.int32, 16)
        himask = np.int32(-65536)

        def unpk(w):
            lo = plsc.bitcast(jnp.left_shift(w, 16), jnp.float32)
            hi = plsc.bitcast(jnp.bitwise_and(w, himask), jnp.float32)
            return lo, hi

        pltpu.sync_copy(kpk_hbm.at[h], kblk)

        def chunk_body(c, _):
            r0 = q0 + c * CHUNK
            pltpu.sync_copy(qpk_hbm.at[h, pl.ds(r0 * DP, CHUNK * DP)], qblk)
            pltpu.sync_copy(idx_hbm.at[pl.ds(r0 * UP, CHUNK * UP)], idxblk)

            def q_body(i, _):
                qbase = jnp.full((16,), i * DP, jnp.int32)
                for sv in range(3):
                    scol = iota16 + (i * UP + sv * 16)
                    idxv = plsc.load_gather(idxblk, [scol])
                    kbase = idxv * DP
                    acc_e = jnp.zeros((16,), jnp.float32)
                    acc_o = jnp.zeros((16,), jnp.float32)
                    for dp in range(DP):
                        kw = plsc.load_gather(kblk, [kbase + dp])
                        qw = plsc.load_gather(qblk, [qbase + dp])
                        k_e, k_o = unpk(kw)
                        q_e, q_o = unpk(qw)
                        acc_e = acc_e + k_e * q_e
                        acc_o = acc_o + k_o * q_o
                    plsc.store_scatter(acc, [scol], acc_e + acc_o)
                return 0

            lax.fori_loop(0, CHUNK, q_body, 0)

            def fin_body(g, _):
                rows = iota16 + g * 16
                base = rows * UP
                mx = jnp.full((16,), NEG, jnp.float32)
                sm = jnp.zeros((16,), jnp.float32)
                for s in range(U):
                    vals = plsc.load_gather(acc, [base + s])
                    mx = jnp.maximum(mx, vals)
                    sm = sm + vals
                plsc.store_scatter(mtile, [rows + c * CHUNK], mx - sm * np.float32(1.0 / L))
                return 0

            lax.fori_loop(0, CHUNK // 16, fin_body, 0)
            return 0

        lax.fori_loop(0, QPT // CHUNK, chunk_body, 0)
        pltpu.sync_copy(mtile, m_hbm.at[h, pl.ds(q0, QPT)])

    return sc_kernel(qpk, kpk, idxp)


def _pack_pairs(x16):
    bits = lax.bitcast_convert_type(x16, jnp.uint16).astype(jnp.uint32)
    word = bits[..., 0::2] | (bits[..., 1::2] << 16)
    return word.astype(jnp.int32).reshape(H, L * DP)


def _tc_body(m_ref, q_ref, k_ref, v_ref, out_ref, qred, upds, ctx, idxs):
    mrow = m_ref[0]
    iota = lax.broadcasted_iota(jnp.int32, (1, L), 1)
    qred[pl.ds(U, 8), :] = jnp.zeros((8, D), jnp.float32)

    def tk(t, mcur):
        mx = jnp.max(mcur)
        it = jnp.min(jnp.where(mcur == mx, iota, L)).astype(jnp.int32)
        idxs[t] = it
        qred[pl.ds(t, 1), :] = q_ref[0, pl.ds(it, 1), :]
        return jnp.where(iota == it, NEG, mcur)

    lax.fori_loop(0, U, tk, mrow)

    kk = k_ref[0]
    scores = lax.dot_general(
        qred[...].astype(jnp.bfloat16), kk.astype(jnp.bfloat16),
        (((1,), (1,)), ((), ())),
        preferred_element_type=jnp.float32,
    ) * np.float32(0.125)
    smax = jnp.max(scores, axis=1, keepdims=True)
    e = jnp.exp(scores - smax)
    attn = e / jnp.sum(e, axis=1, keepdims=True)
    vv = v_ref[0]
    upds[...] = lax.dot_general(
        attn.astype(jnp.bfloat16), vv.astype(jnp.bfloat16),
        (((1,), (0,)), ((), ())),
        preferred_element_type=jnp.float32,
    )
    vmean = jnp.mean(vv, axis=0, keepdims=True)
    ctx[...] = jnp.broadcast_to(vmean, (U + 8, D))

    def sc(k, _):
        slot = jnp.minimum(idxs[k], U - 1)
        ctx[pl.ds(slot, 1), :] = upds[pl.ds(k, 1), :]
        return 0

    lax.fori_loop(0, U, sc, 0)
    out_ref[0] = ctx[0:U, :]


def _tc_stage2(m, qT, kT, vT):
    return pl.pallas_call(
        _tc_body,
        grid=(H,),
        in_specs=[
            pl.BlockSpec((1, 1, L), lambda h: (h, 0, 0)),
            pl.BlockSpec((1, L, D), lambda h: (h, 0, 0)),
            pl.BlockSpec((1, L, D), lambda h: (h, 0, 0)),
            pl.BlockSpec((1, L, D), lambda h: (h, 0, 0)),
        ],
        out_specs=pl.BlockSpec((1, U, D), lambda h: (h, 0, 0)),
        out_shape=jax.ShapeDtypeStruct((H, U, D), jnp.float32),
        scratch_shapes=[
            pltpu.VMEM((U + 8, D), jnp.float32),
            pltpu.VMEM((U + 8, D), jnp.float32),
            pltpu.VMEM((U + 8, D), jnp.float32),
            pltpu.SMEM((U,), jnp.int32),
        ],
    )(m, qT, kT, vT)


def kernel(queries, keys, values, attn_mask):
    q16 = jnp.transpose(queries[0], (1, 0, 2)).astype(jnp.bfloat16)
    k16 = jnp.transpose(keys[0], (1, 0, 2)).astype(jnp.bfloat16)
    qT = q16.astype(jnp.float32)
    kT = k16.astype(jnp.float32)
    vT = jnp.transpose(values[0], (1, 0, 2))
    idx = jax.random.randint(jax.random.key(42), (L, U), 0, L)
    idxp = jnp.concatenate(
        [idx, jnp.zeros((L, UP - U), idx.dtype)], axis=1
    ).astype(jnp.int32)
    m = _sc_stage1(_pack_pairs(q16), _pack_pairs(k16), idxp.reshape(L * UP))
    out3 = _tc_stage2(m.reshape(H, 1, L), qT, kT, vT)
    return (jnp.transpose(out3, (1, 0, 2))[None], None)


# speedup vs baseline: 1.5767x; 1.4779x over previous
import functools

import jax
import numpy as np
import jax.numpy as jnp
from jax import lax
from jax.experimental import pallas as pl
from jax.experimental.pallas import tpu as pltpu
from jax.experimental.pallas import tpu_sc as plsc

L = 2048
H = 16
D = 64
U = 40
UP = 48
DH = 32
CHUNK = 256
QPT = 1024
NEG = np.float32(-3.0e38)


DP = D // 2
KS = DP + 1
AS = UP + 1


def _sc_stage1(qpk, kpk, idxp):
    mesh = plsc.VectorSubcoreMesh(core_axis_name="c", subcore_axis_name="s")

    @functools.partial(
        pl.kernel,
        mesh=mesh,
        compiler_params=pltpu.CompilerParams(needs_layout_passes=False),
        out_type=jax.ShapeDtypeStruct((H, L), jnp.float32),
        scratch_types=[
            pltpu.VMEM((L * KS,), jnp.int32),
            pltpu.VMEM((CHUNK * DP,), jnp.int32),
            pltpu.VMEM((CHUNK * UP,), jnp.int32),
            pltpu.VMEM((CHUNK * AS,), jnp.float32),
            pltpu.VMEM((QPT,), jnp.float32),
        ],
    )
    def sc_kernel(qpk_hbm, kpk_hbm, idx_hbm, m_hbm, kblk, qblk, idxblk, acc, mtile):
        h = lax.axis_index("s")
        half = lax.axis_index("c")
        q0 = half * QPT
        iota16 = lax.iota(jnp.int32, 16)
        himask = np.int32(-65536)

        def unpk(w):
            lo = plsc.bitcast(jnp.left_shift(w, 16), jnp.float32)
            hi = plsc.bitcast(jnp.bitwise_and(w, himask), jnp.float32)
            return lo, hi

        pltpu.sync_copy(kpk_hbm.at[h], kblk)

        def chunk_body(c, _):
            r0 = q0 + c * CHUNK
            pltpu.sync_copy(qpk_hbm.at[h, pl.ds(r0 * DP, CHUNK * DP)], qblk)
            pltpu.sync_copy(idx_hbm.at[pl.ds(r0 * UP, CHUNK * UP)], idxblk)

            def q_body(i, _):
                qw0 = plsc.load_gather(qblk, [iota16 + i * DP])
                qw1 = plsc.load_gather(qblk, [iota16 + (i * DP + 16)])
                qws = (qw0, qw1)
                for sv in range(3):
                    scol = iota16 + (i * UP + sv * 16)
                    idxv = plsc.load_gather(idxblk, [scol])
                    kbase = idxv * KS
                    acc_e = jnp.zeros((16,), jnp.float32)
                    acc_o = jnp.zeros((16,), jnp.float32)
                    for dp in range(DP):
                        kw = plsc.load_gather(kblk, [kbase + dp])
                        qw = qws[dp // 16].at[jnp.full((16,), dp % 16, jnp.int32)].get(
                            mode="promise_in_bounds")
                        k_e, k_o = unpk(kw)
                        q_e, q_o = unpk(qw)
                        acc_e = acc_e + k_e * q_e
                        acc_o = acc_o + k_o * q_o
                    plsc.store_scatter(acc, [iota16 + (i * AS + sv * 16)], acc_e + acc_o)
                return 0

            lax.fori_loop(0, CHUNK, q_body, 0)

            def fin_body(g, _):
                rows = iota16 + g * 16
                base = rows * AS
                mx = jnp.full((16,), NEG, jnp.float32)
                sm = jnp.zeros((16,), jnp.float32)
                for s in range(U):
                    vals = plsc.load_gather(acc, [base + s])
                    mx = jnp.maximum(mx, vals)
                    sm = sm + vals
                plsc.store_scatter(mtile, [rows + c * CHUNK], mx - sm * np.float32(1.0 / L))
                return 0

            lax.fori_loop(0, CHUNK // 16, fin_body, 0)
            return 0

        lax.fori_loop(0, QPT // CHUNK, chunk_body, 0)
        pltpu.sync_copy(mtile, m_hbm.at[h, pl.ds(q0, QPT)])

    return sc_kernel(qpk, kpk, idxp)


def _pack_pairs(x16, stride):
    bits = lax.bitcast_convert_type(x16, jnp.uint16).astype(jnp.uint32)
    word = bits[..., 0::2] | (bits[..., 1::2] << 16)
    if stride > DP:
        word = jnp.pad(word, ((0, 0), (0, 0), (0, stride - DP)))
    return word.astype(jnp.int32).reshape(H, L * stride)


def _tc_body(m_ref, q_ref, k_ref, v_ref, out_ref, qred, upds, ctx, idxs):
    mrow = m_ref[0]
    iota = lax.broadcasted_iota(jnp.int32, (1, L), 1)
    qred[pl.ds(U, 8), :] = jnp.zeros((8, D), jnp.float32)

    def tk(t, mcur):
        mx = jnp.max(mcur)
        it = jnp.min(jnp.where(mcur == mx, iota, L)).astype(jnp.int32)
        idxs[t] = it
        qred[pl.ds(t, 1), :] = q_ref[0, pl.ds(it, 1), :]
        return jnp.where(iota == it, NEG, mcur)

    lax.fori_loop(0, U, tk, mrow)

    kk = k_ref[0]
    scores = lax.dot_general(
        qred[...].astype(jnp.bfloat16), kk.astype(jnp.bfloat16),
        (((1,), (1,)), ((), ())),
        preferred_element_type=jnp.float32,
    ) * np.float32(0.125)
    smax = jnp.max(scores, axis=1, keepdims=True)
    e = jnp.exp(scores - smax)
    attn = e / jnp.sum(e, axis=1, keepdims=True)
    vv = v_ref[0]
    upds[...] = lax.dot_general(
        attn.astype(jnp.bfloat16), vv.astype(jnp.bfloat16),
        (((1,), (0,)), ((), ())),
        preferred_element_type=jnp.float32,
    )
    vmean = jnp.mean(vv, axis=0, keepdims=True)
    ctx[...] = jnp.broadcast_to(vmean, (U + 8, D))

    def sc(k, _):
        slot = jnp.minimum(idxs[k], U - 1)
        ctx[pl.ds(slot, 1), :] = upds[pl.ds(k, 1), :]
        return 0

    lax.fori_loop(0, U, sc, 0)
    out_ref[0] = ctx[0:U, :]


def _tc_stage2(m, qT, kT, vT):
    return pl.pallas_call(
        _tc_body,
        grid=(H,),
        in_specs=[
            pl.BlockSpec((1, 1, L), lambda h: (h, 0, 0)),
            pl.BlockSpec((1, L, D), lambda h: (h, 0, 0)),
            pl.BlockSpec((1, L, D), lambda h: (h, 0, 0)),
            pl.BlockSpec((1, L, D), lambda h: (h, 0, 0)),
        ],
        out_specs=pl.BlockSpec((1, U, D), lambda h: (h, 0, 0)),
        out_shape=jax.ShapeDtypeStruct((H, U, D), jnp.float32),
        scratch_shapes=[
            pltpu.VMEM((U + 8, D), jnp.float32),
            pltpu.VMEM((U + 8, D), jnp.float32),
            pltpu.VMEM((U + 8, D), jnp.float32),
            pltpu.SMEM((U,), jnp.int32),
        ],
    )(m, qT, kT, vT)


def kernel(queries, keys, values, attn_mask):
    q16 = jnp.transpose(queries[0], (1, 0, 2)).astype(jnp.bfloat16)
    k16 = jnp.transpose(keys[0], (1, 0, 2)).astype(jnp.bfloat16)
    qT = q16.astype(jnp.float32)
    kT = k16.astype(jnp.float32)
    vT = jnp.transpose(values[0], (1, 0, 2))
    idx = jax.random.randint(jax.random.key(42), (L, U), 0, L)
    idxp = jnp.concatenate(
        [idx, jnp.zeros((L, UP - U), idx.dtype)], axis=1
    ).astype(jnp.int32)
    m = _sc_stage1(_pack_pairs(q16, DP), _pack_pairs(k16, KS), idxp.reshape(L * UP))
    out3 = _tc_stage2(m.reshape(H, 1, L), qT, kT, vT)
    return (jnp.transpose(out3, (1, 0, 2))[None], None)
